# baseline (speedup 1.0000x reference)
import functools

import jax
import jax.numpy as jnp
from jax import lax
from jax.experimental import pallas as pl
from jax.experimental.pallas import tpu as pltpu
from jax.experimental.pallas import tpu_sc as plsc

VOCAB_SIZE = 32000
COPY_WIDTH = 256
N_TOKENS = 1024
EPS = 1e-10

NUM_CORES = 2
NUM_SUBCORES = 16
LANES = 16
NUM_WORKERS = NUM_CORES * NUM_SUBCORES
TOK_PER_WORKER = N_TOKENS // NUM_WORKERS
CHUNKS = TOK_PER_WORKER // LANES

_LN2 = 0.6931471805599453
_SQRT2 = 1.4142135623730951

_GATHER_DNUMS = lax.GatherDimensionNumbers(
    offset_dims=(), collapsed_slice_dims=(0,), start_index_map=(0,))


def _log_f32(x):
    bits = lax.bitcast_convert_type(x, jnp.int32)
    e = lax.shift_right_arithmetic(bits, 23) - 127
    mbits = lax.bitwise_or(lax.bitwise_and(bits, 0x007FFFFF), 0x3F800000)
    m = lax.bitcast_convert_type(mbits, jnp.float32)
    adj = m > _SQRT2
    m = jnp.where(adj, m * 0.5, m)
    ef = e.astype(jnp.float32) + jnp.where(adj, 1.0, 0.0)
    t = (m - 1.0) / (m + 1.0)
    t2 = t * t
    p = t * (2.0 + t2 * (2.0 / 3.0 + t2 * (0.4 + t2 * (2.0 / 7.0 + t2 * (2.0 / 9.0)))))
    return ef * _LN2 + p


def _dyn_gather16(vec, idx):
    return lax.gather(
        vec, idx[:, None], _GATHER_DNUMS, (1,),
        indices_are_sorted=False, unique_indices=False,
        mode=lax.GatherScatterMode.PROMISE_IN_BOUNDS)


_MESH = plsc.VectorSubcoreMesh(
    core_axis_name="c", subcore_axis_name="s",
    num_cores=NUM_CORES, num_subcores=NUM_SUBCORES,
)


@functools.partial(
    pl.kernel,
    out_type=jax.ShapeDtypeStruct((N_TOKENS,), jnp.float32),
    mesh=_MESH,
    scratch_types=[
        pltpu.VMEM((TOK_PER_WORKER,), jnp.int32),
        pltpu.VMEM((TOK_PER_WORKER,), jnp.int32),
        pltpu.VMEM((TOK_PER_WORKER,), jnp.int32),
        pltpu.VMEM((TOK_PER_WORKER, 8, 128), jnp.float32),
        pltpu.VMEM((TOK_PER_WORKER,), jnp.float32),
        pltpu.VMEM((TOK_PER_WORKER,), jnp.float32),
        pltpu.SemaphoreType.DMA,
        pltpu.SemaphoreType.DMA,
    ],
)
def _loss_kernel(outp_hbm, copyp_hbm, align_hbm, target_hbm, out_hbm,
                 align_v, target_v, cidx_v, ovg_v, cv_v, loss_v,
                 sem_o, sem_c):
    wid = lax.axis_index("s") * NUM_CORES + lax.axis_index("c")
    base = wid * TOK_PER_WORKER

    pltpu.sync_copy(align_hbm.at[pl.ds(base, TOK_PER_WORKER)], align_v)
    pltpu.sync_copy(target_hbm.at[pl.ds(base, TOK_PER_WORKER)], target_v)

    lanes = lax.iota(jnp.int32, LANES)
    for j in range(CHUNKS):
        sl = pl.ds(j * LANES, LANES)
        row = base + j * LANES + lanes
        cidx_v[sl] = row * COPY_WIDTH + align_v[sl]

    cp_c = pltpu.async_copy(copyp_hbm.at[cidx_v], cv_v, sem_c)

    copies = []
    for j in range(CHUNKS):
        col16 = lax.bitwise_and(target_v[pl.ds(j * LANES, LANES)], -128)
        for k in range(LANES):
            tok = j * LANES + k
            row0 = pl.multiple_of(base + (tok & -8), 8)
            col0 = pl.multiple_of(col16[k], 128)
            copies.append(pltpu.async_copy(
                outp_hbm.at[pl.ds(row0, 8), pl.ds(col0, 128)],
                ovg_v.at[tok], sem_o))
    for cp in copies:
        cp.wait()
    cp_c.wait()

    for j in range(CHUNKS):
        sl = pl.ds(j * LANES, LANES)
        av = align_v[sl]
        tv = target_v[sl]

        sub16 = lax.bitwise_and(tv, 112)
        low4 = lax.bitwise_and(tv, 15)
        vocab_p = jnp.zeros((LANES,), jnp.float32)
        for k in range(LANES):
            tok = j * LANES + k
            off = pl.multiple_of(sub16[k], 8)
            v16 = ovg_v[tok, tok % 8, pl.ds(off, LANES)]
            g = _dyn_gather16(v16, jnp.full((LANES,), low4[k], jnp.int32))
            vocab_p = jnp.where(lanes == k, g, vocab_p)

        copy_p = cv_v[sl]
        copy_tok = jnp.where(av == 0, 0.0, copy_p) + EPS
        non_copy = (av == 0) | (tv != 0)
        probs = jnp.where(non_copy, copy_tok + vocab_p, copy_tok)
        loss = -_log_f32(probs + EPS)
        loss_v[sl] = jnp.where(tv == 0, 0.0, loss)

    pltpu.sync_copy(loss_v, out_hbm.at[pl.ds(base, TOK_PER_WORKER)])


def kernel(out_prob, copy_prob, align, target, src_tgt_map, label_smoothing):
    del src_tgt_map, label_smoothing
    flat_copy = copy_prob.reshape(-1)
    flat_align = align.reshape(-1).astype(jnp.int32)
    flat_target = target.reshape(-1).astype(jnp.int32)
    return _loss_kernel(out_prob, flat_copy, flat_align, flat_target)


# speedup vs baseline: 1.1116x; 1.1116x over previous
import functools

import jax
import jax.numpy as jnp
from jax import lax
from jax.experimental import pallas as pl
from jax.experimental.pallas import tpu as pltpu
from jax.experimental.pallas import tpu_sc as plsc

VOCAB_SIZE = 32000
COPY_WIDTH = 256
N_TOKENS = 1024
EPS = 1e-10

NUM_CORES = 2
NUM_SUBCORES = 16
LANES = 16
NUM_WORKERS = NUM_CORES * NUM_SUBCORES
TOK_PER_WORKER = N_TOKENS // NUM_WORKERS
CHUNKS = TOK_PER_WORKER // LANES

_LN2 = 0.6931471805599453
_SQRT2 = 1.4142135623730951

_GATHER_DNUMS = lax.GatherDimensionNumbers(
    offset_dims=(), collapsed_slice_dims=(0,), start_index_map=(0,))


def _log_f32(x):
    bits = lax.bitcast_convert_type(x, jnp.int32)
    e = lax.shift_right_arithmetic(bits, 23) - 127
    mbits = lax.bitwise_or(lax.bitwise_and(bits, 0x007FFFFF), 0x3F800000)
    m = lax.bitcast_convert_type(mbits, jnp.float32)
    adj = m > _SQRT2
    m = jnp.where(adj, m * 0.5, m)
    ef = e.astype(jnp.float32) + jnp.where(adj, 1.0, 0.0)
    t = (m - 1.0) / (m + 1.0)
    t2 = t * t
    p = t * (2.0 + t2 * (2.0 / 3.0 + t2 * (0.4 + t2 * (2.0 / 7.0 + t2 * (2.0 / 9.0)))))
    return ef * _LN2 + p


def _dyn_gather16(vec, idx):
    return lax.gather(
        vec, idx[:, None], _GATHER_DNUMS, (1,),
        indices_are_sorted=False, unique_indices=False,
        mode=lax.GatherScatterMode.PROMISE_IN_BOUNDS)


_MESH = plsc.VectorSubcoreMesh(
    core_axis_name="c", subcore_axis_name="s",
    num_cores=NUM_CORES, num_subcores=NUM_SUBCORES,
)


@functools.partial(
    pl.kernel,
    out_type=jax.ShapeDtypeStruct((N_TOKENS,), jnp.float32),
    mesh=_MESH,
    scratch_types=[
        pltpu.VMEM((TOK_PER_WORKER,), jnp.int32),
        pltpu.VMEM((TOK_PER_WORKER,), jnp.int32),
        pltpu.VMEM((TOK_PER_WORKER, 8, 128), jnp.float32),
        pltpu.VMEM((TOK_PER_WORKER, COPY_WIDTH), jnp.float32),
        pltpu.VMEM((TOK_PER_WORKER,), jnp.float32),
        pltpu.SemaphoreType.DMA,
        pltpu.SemaphoreType.DMA,
        pltpu.SemaphoreType.DMA,
    ],
)
def _loss_kernel(outp_hbm, copyp_hbm, at_hbm, out_hbm,
                 align_v, target_v, ovg_v, ovc_v, loss_v,
                 sem_i, sem_o, sem_c):
    wid = lax.axis_index("s") * NUM_CORES + lax.axis_index("c")
    base = wid * TOK_PER_WORKER

    cp_a = pltpu.async_copy(at_hbm.at[pl.ds(base, TOK_PER_WORKER)],
                            align_v, sem_i)
    cp_t = pltpu.async_copy(at_hbm.at[pl.ds(N_TOKENS + base, TOK_PER_WORKER)],
                            target_v, sem_i)
    row0w = pl.multiple_of(base, 8)
    cp_slab = pltpu.async_copy(copyp_hbm.at[pl.ds(row0w, TOK_PER_WORKER)],
                               ovc_v, sem_c)
    cp_a.wait()
    cp_t.wait()

    lanes = lax.iota(jnp.int32, LANES)

    copies = []
    for j in range(CHUNKS):
        col16 = lax.bitwise_and(target_v[pl.ds(j * LANES, LANES)], -128)
        for k in range(LANES):
            tok = j * LANES + k
            row0 = pl.multiple_of(base + (tok & -8), 8)
            col0 = pl.multiple_of(col16[k], 128)
            copies.append(pltpu.async_copy(
                outp_hbm.at[pl.ds(row0, 8), pl.ds(col0, 128)],
                ovg_v.at[tok], sem_o))
    for cp in copies:
        cp.wait()
    cp_slab.wait()

    for j in range(CHUNKS):
        sl = pl.ds(j * LANES, LANES)
        av = align_v[sl]
        tv = target_v[sl]

        tsub = lax.bitwise_and(tv, 112)
        tlow = lax.bitwise_and(tv, 15)
        asub = lax.bitwise_and(av, 240)
        alow = lax.bitwise_and(av, 15)
        vocab_p = jnp.zeros((LANES,), jnp.float32)
        copy_p = jnp.zeros((LANES,), jnp.float32)
        for k in range(LANES):
            tok = j * LANES + k
            voff = pl.multiple_of(tsub[k], 16)
            v16 = ovg_v[tok, tok % 8, pl.ds(voff, LANES)]
            gv = _dyn_gather16(v16, jnp.full((LANES,), tlow[k], jnp.int32))
            vocab_p = jnp.where(lanes == k, gv, vocab_p)
            coff = pl.multiple_of(asub[k], 16)
            c16 = ovc_v[tok, pl.ds(coff, LANES)]
            gc = _dyn_gather16(c16, jnp.full((LANES,), alow[k], jnp.int32))
            copy_p = jnp.where(lanes == k, gc, copy_p)

        copy_tok = jnp.where(av == 0, 0.0, copy_p) + EPS
        non_copy = (av == 0) | (tv != 0)
        probs = jnp.where(non_copy, copy_tok + vocab_p, copy_tok)
        loss = -_log_f32(probs + EPS)
        loss_v[sl] = jnp.where(tv == 0, 0.0, loss)

    pltpu.sync_copy(loss_v, out_hbm.at[pl.ds(base, TOK_PER_WORKER)])


def kernel(out_prob, copy_prob, align, target, src_tgt_map, label_smoothing):
    del src_tgt_map, label_smoothing
    at = jnp.concatenate([
        align.reshape(-1).astype(jnp.int32),
        target.reshape(-1).astype(jnp.int32),
    ])
    return _loss_kernel(out_prob, copy_prob, at)
